# baseline (speedup 1.0000x reference)
import jax
import jax.numpy as jnp
from jax import lax
from jax.experimental import pallas as pl
from jax.experimental.pallas import tpu as pltpu
from jax.experimental.pallas import tpu_sc as plsc

_V = 1000000
_VT = 7813
_VP = _VT * 128
_BATCH = 16384
_K = 16
_KP = _K // 2
_NC = 2
_NS = 16
_NW = _NC * _NS
_BPW = _BATCH // _NW
_CHUNK = 128
_NCHUNK = _BPW // _CHUNK
_NFCH = _BPW * _KP // _CHUNK
_GROUPS = _BPW // _K


def _mf_sc_kernel(uidx_hbm, vidx_hbm, w_hbm, h_hbm, out_hbm,
                  uidx_v, vidx_v, fidx_v, gidx_v, urows_v, vrows_v, out_v,
                  sem):
    wid = lax.axis_index("s") * _NC + lax.axis_index("c")
    base = wid * _BPW

    pltpu.sync_copy(uidx_hbm.at[pl.ds(wid * _NCHUNK, _NCHUNK)], uidx_v)
    pltpu.sync_copy(vidx_hbm.at[pl.ds(wid * _NCHUNK, _NCHUNK)], vidx_v)

    def idx_body(cp, carry):
        cbase = cp * 128
        for j in range(_NCHUNK):
            for s in range(_CHUNK // _K):
                ids = uidx_v[j, pl.ds(s * _K, _K)]
                gds = vidx_v[j, pl.ds(s * _K, _K)]
                fidx_v[cp * _NCHUNK + j, pl.ds(s * _K, _K)] = (
                    cbase + (ids >> 7) * 1024 + (ids & 127))
                gidx_v[cp * _NCHUNK + j, pl.ds(s * _K, _K)] = (
                    cbase + (gds >> 7) * 1024 + (gds & 127))
        return carry

    lax.fori_loop(0, _KP, idx_body, 0)

    copies = []
    for j in range(_NFCH):
        copies.append(pltpu.async_copy(
            w_hbm.at[fidx_v.at[j]], urows_v.at[pl.ds(j * _CHUNK, _CHUNK)],
            sem))
        copies.append(pltpu.async_copy(
            h_hbm.at[gidx_v.at[j]], vrows_v.at[pl.ds(j * _CHUNK, _CHUNK)],
            sem))
    for cp in copies:
        cp.wait()

    mask_hi = jnp.full((_K,), -65536, jnp.int32)

    def group_body(g, carry):
        acc = jnp.zeros((_K,), jnp.float32)
        for cp in range(_KP):
            off = cp * _BPW + g * _K
            uw = plsc.bitcast(urows_v[pl.ds(off, _K)], jnp.int32)
            vw = plsc.bitcast(vrows_v[pl.ds(off, _K)], jnp.int32)
            ulo = plsc.bitcast(uw << 16, jnp.float32)
            vlo = plsc.bitcast(vw << 16, jnp.float32)
            uhi = plsc.bitcast(uw & mask_hi, jnp.float32)
            vhi = plsc.bitcast(vw & mask_hi, jnp.float32)
            acc = acc + ulo * vlo + uhi * vhi
        out_v[pl.ds(g * _K, _K)] = 1.0 / (1.0 + jnp.exp(-acc))
        return carry

    lax.fori_loop(0, _GROUPS, group_body, 0)

    pltpu.sync_copy(out_v, out_hbm.at[pl.ds(base, _BPW)])


def _packed_flat(t):
    tb = jnp.pad(t.astype(jnp.bfloat16), ((0, _VP - _V), (0, 0)))
    flat_bf = (tb.reshape(_VT, 128, 2, _KP)
               .transpose(0, 3, 1, 2)
               .reshape(_VT * _KP * _CHUNK * 2))
    return jax.lax.bitcast_convert_type(
        flat_bf.reshape(_VT * _KP * _CHUNK, 2), jnp.float32)


@jax.jit
def kernel(x, W, H):
    uidx = x[:, 0].reshape(_NW * _NCHUNK, _CHUNK)
    vidx = x[:, 1].reshape(_NW * _NCHUNK, _CHUNK)
    wf = _packed_flat(W)
    hf = _packed_flat(H)
    mesh = plsc.VectorSubcoreMesh(core_axis_name="c", subcore_axis_name="s")
    run = pl.kernel(
        _mf_sc_kernel,
        out_type=jax.ShapeDtypeStruct((_BATCH,), jnp.float32),
        mesh=mesh,
        scratch_types=[
            pltpu.VMEM((_NCHUNK, _CHUNK), jnp.int32),
            pltpu.VMEM((_NCHUNK, _CHUNK), jnp.int32),
            pltpu.VMEM((_NFCH, _CHUNK), jnp.int32),
            pltpu.VMEM((_NFCH, _CHUNK), jnp.int32),
            pltpu.VMEM((_BPW * _KP,), jnp.float32),
            pltpu.VMEM((_BPW * _KP,), jnp.float32),
            pltpu.VMEM((_BPW,), jnp.float32),
            pltpu.SemaphoreType.DMA,
        ],
        compiler_params=pltpu.CompilerParams(
            needs_layout_passes=False, use_tc_tiling_on_sc=False),
    )
    return run(uidx, vidx, wf, hf)


# speedup vs baseline: 1.0973x; 1.0973x over previous
import jax
import jax.numpy as jnp
from jax import lax
from jax.experimental import pallas as pl
from jax.experimental.pallas import tpu as pltpu
from jax.experimental.pallas import tpu_sc as plsc

_V = 1000000
_VT = 7813
_VP = _VT * 128
_BATCH = 16384
_K = 16
_KP = _K // 2
_NC = 2
_NS = 16
_NW = _NC * _NS
_BPW = _BATCH // _NW
_CHUNK = 128
_NCHUNK = _BPW // _CHUNK
_NFCH = _BPW * _KP // _CHUNK
_GROUPS = _BPW // _K


def _mf_sc_kernel(uidx_hbm, vidx_hbm, w_hbm, h_hbm, out_hbm,
                  uidx_v, vidx_v, fidx_v, gidx_v, urows_v, vrows_v, out_v,
                  sem):
    wid = lax.axis_index("s") * _NC + lax.axis_index("c")
    base = wid * _BPW

    pltpu.sync_copy(uidx_hbm.at[pl.ds(wid * _NCHUNK, _NCHUNK)], uidx_v)
    pltpu.sync_copy(vidx_hbm.at[pl.ds(wid * _NCHUNK, _NCHUNK)], vidx_v)

    def idx_body(cp, carry):
        cbase = cp * 128
        for j in range(_NCHUNK):
            for s in range(_CHUNK // _K):
                ids = uidx_v[j, pl.ds(s * _K, _K)]
                gds = vidx_v[j, pl.ds(s * _K, _K)]
                fidx_v[cp * _NCHUNK + j, pl.ds(s * _K, _K)] = (
                    cbase + (ids >> 7) * 1024 + (ids & 127))
                gidx_v[cp * _NCHUNK + j, pl.ds(s * _K, _K)] = (
                    cbase + (gds >> 7) * 1024 + (gds & 127))
        return carry

    lax.fori_loop(0, _KP, idx_body, 0)

    copies = []
    for j in range(_NFCH):
        copies.append(pltpu.async_copy(
            w_hbm.at[fidx_v.at[j]], urows_v.at[pl.ds(j * _CHUNK, _CHUNK)],
            sem))
        copies.append(pltpu.async_copy(
            h_hbm.at[gidx_v.at[j]], vrows_v.at[pl.ds(j * _CHUNK, _CHUNK)],
            sem))
    for cp in copies:
        cp.wait()

    mask_hi = jnp.full((_K,), -65536, jnp.int32)

    def group_body(g, carry):
        acc = jnp.zeros((_K,), jnp.float32)
        for cp in range(_KP):
            off = cp * _BPW + g * _K
            uw = plsc.bitcast(urows_v[pl.ds(off, _K)], jnp.int32)
            vw = plsc.bitcast(vrows_v[pl.ds(off, _K)], jnp.int32)
            ulo = plsc.bitcast(uw << 16, jnp.float32)
            vlo = plsc.bitcast(vw << 16, jnp.float32)
            uhi = plsc.bitcast(uw & mask_hi, jnp.float32)
            vhi = plsc.bitcast(vw & mask_hi, jnp.float32)
            acc = acc + ulo * vlo + uhi * vhi
        out_v[pl.ds(g * _K, _K)] = 1.0 / (1.0 + jnp.exp(-acc))
        return carry

    lax.fori_loop(0, _GROUPS, group_body, 0)

    pltpu.sync_copy(out_v, out_hbm.at[pl.ds(base, _BPW)])


def _packed_flat(t):
    ti = jax.lax.bitcast_convert_type(t, jnp.uint32)
    words = (ti[:, :_KP] >> 16) | (ti[:, _KP:] & jnp.uint32(0xFFFF0000))
    wp = jnp.pad(words, ((0, _VP - _V), (0, 0)))
    flat = wp.reshape(_VT, 128, _KP).transpose(0, 2, 1).reshape(
        _VT * _KP * _CHUNK)
    return jax.lax.bitcast_convert_type(flat, jnp.float32)


@jax.jit
def kernel(x, W, H):
    uidx = x[:, 0].reshape(_NW * _NCHUNK, _CHUNK)
    vidx = x[:, 1].reshape(_NW * _NCHUNK, _CHUNK)
    wf = _packed_flat(W)
    hf = _packed_flat(H)
    mesh = plsc.VectorSubcoreMesh(core_axis_name="c", subcore_axis_name="s")
    run = pl.kernel(
        _mf_sc_kernel,
        out_type=jax.ShapeDtypeStruct((_BATCH,), jnp.float32),
        mesh=mesh,
        scratch_types=[
            pltpu.VMEM((_NCHUNK, _CHUNK), jnp.int32),
            pltpu.VMEM((_NCHUNK, _CHUNK), jnp.int32),
            pltpu.VMEM((_NFCH, _CHUNK), jnp.int32),
            pltpu.VMEM((_NFCH, _CHUNK), jnp.int32),
            pltpu.VMEM((_BPW * _KP,), jnp.float32),
            pltpu.VMEM((_BPW * _KP,), jnp.float32),
            pltpu.VMEM((_BPW,), jnp.float32),
            pltpu.SemaphoreType.DMA,
        ],
        compiler_params=pltpu.CompilerParams(
            needs_layout_passes=False, use_tc_tiling_on_sc=False),
    )
    return run(uidx, vidx, wf, hf)
